# baseline (speedup 1.0000x reference)
import functools

import jax
import jax.numpy as jnp
from jax import lax
from jax.experimental import pallas as pl
from jax.experimental.pallas import tpu as pltpu
from jax.experimental.pallas import tpu_sc as plsc

ROWS = 2048
W = 32768
NV = W // 16
K = 512
HBITS = 12
HB = 1 << HBITS
SHIFT = 31 - HBITS
NW = 32
RPW = ROWS // NW

_mesh = plsc.VectorSubcoreMesh(core_axis_name="c", subcore_axis_name="s")


def _scalar(v):
    return jnp.reshape(lax.slice(v, (0,), (1,)), ())


@functools.partial(
    pl.kernel,
    out_type=jax.ShapeDtypeStruct((ROWS, W), jnp.float32),
    mesh=_mesh,
    compiler_params=pltpu.CompilerParams(needs_layout_passes=False),
    scratch_types=[
        pltpu.VMEM((W,), jnp.float32),
        pltpu.VMEM((HB + 16,), jnp.int32),
        pltpu.VMEM((W + 16,), jnp.int32),
    ],
)
def _sc_topk(x_hbm, o_hbm, row_v, hist_v, cand_v):
    wid = lax.axis_index("s") * 2 + lax.axis_index("c")
    ones16 = jnp.ones((16,), jnp.int32)
    zero16i = jnp.zeros((16,), jnp.int32)
    zero16f = jnp.zeros((16,), jnp.float32)

    def do_row(rr, carry):
        row = wid * RPW + rr
        pltpu.sync_copy(x_hbm.at[row], row_v)

        def zb(i, c):
            hist_v[pl.ds(i * 16, 16)] = zero16i
            return c
        lax.fori_loop(0, HB // 16, zb, 0)

        def h1(i, c):
            v = row_v[pl.ds(i * 16, 16)]
            keys = plsc.bitcast(v, jnp.int32) & jnp.int32(0x7FFFFFFF)
            plsc.addupdate_scatter(hist_v, (keys >> SHIFT,), ones16)
            return c
        lax.fori_loop(0, NV, h1, 0)

        def hcnt(b):
            return _scalar(hist_v[pl.ds(b, 16)])

        def wcond(bc):
            b, acc = bc
            return acc + hcnt(b) < K

        def wbody(bc):
            b, acc = bc
            return b - 1, acc + hcnt(b)

        bstar, acc = lax.while_loop(wcond, wbody, (jnp.int32(HB - 1), jnp.int32(0)))
        k2 = K - acc

        def cp(i, off):
            v = row_v[pl.ds(i * 16, 16)]
            keys = plsc.bitcast(v, jnp.int32) & jnp.int32(0x7FFFFFFF)
            m = (keys >> SHIFT) == bstar
            plsc.store_compressed(cand_v.at[pl.ds(off, 16)], keys, mask=m)
            return off + _scalar(plsc.all_reduce_population_count(m))
        ncand = lax.fori_loop(0, NV, cp, jnp.int32(0))
        cand_v[pl.ds(ncand, 16)] = zero16i

        nvc = (ncand + 15) >> 4

        def bit_step(j, lo):
            mid = lo | (jnp.int32(1) << (jnp.int32(SHIFT - 1) - j))

            def cnt_body(i, a):
                vk = cand_v[pl.ds(i * 16, 16)]
                return a + jnp.where(vk >= mid, 1, 0)
            av = lax.fori_loop(0, nvc, cnt_body, zero16i)
            return jnp.where(jnp.sum(av) >= k2, mid, lo)
        thr = lax.fori_loop(0, SHIFT, bit_step, bstar << SHIFT)

        def mp(i, c):
            v = row_v[pl.ds(i * 16, 16)]
            keys = plsc.bitcast(v, jnp.int32) & jnp.int32(0x7FFFFFFF)
            row_v[pl.ds(i * 16, 16)] = jnp.where(keys >= thr, v, zero16f)
            return c
        lax.fori_loop(0, NV, mp, 0)

        pltpu.sync_copy(row_v, o_hbm.at[row])
        return carry

    lax.fori_loop(0, RPW, do_row, 0)


def kernel(input):
    x = input
    B, C, _ = x.shape
    out = _sc_topk(x.reshape(ROWS, W))
    return out.reshape(B, C, W)


# speedup vs baseline: 2.0134x; 2.0134x over previous
import functools

import jax
import jax.numpy as jnp
from jax import lax
from jax.experimental import pallas as pl
from jax.experimental.pallas import tpu as pltpu
from jax.experimental.pallas import tpu_sc as plsc

ROWS = 2048
W = 32768
NV = W // 16
K = 512
HBITS = 12
HB = 1 << HBITS
SHIFT = 31 - HBITS
NW = 32
RPW = ROWS // NW

_mesh = plsc.VectorSubcoreMesh(core_axis_name="c", subcore_axis_name="s")


def _scalar(v):
    return jnp.reshape(lax.slice(v, (0,), (1,)), ())


@functools.partial(
    pl.kernel,
    out_type=jax.ShapeDtypeStruct((ROWS, W), jnp.float32),
    mesh=_mesh,
    compiler_params=pltpu.CompilerParams(needs_layout_passes=False),
    scratch_types=[
        pltpu.VMEM((W,), jnp.float32),
        pltpu.VMEM((HB + 16,), jnp.int32),
        pltpu.VMEM((W + 16,), jnp.int32),
    ],
)
def _sc_topk(x_hbm, o_hbm, row_v, hist_v, cand_v):
    wid = lax.axis_index("s") * 2 + lax.axis_index("c")
    ones16 = jnp.ones((16,), jnp.int32)
    zero16i = jnp.zeros((16,), jnp.int32)
    zero16f = jnp.zeros((16,), jnp.float32)

    def do_row(rr, carry):
        row = wid * RPW + rr
        pltpu.sync_copy(x_hbm.at[row], row_v)

        @plsc.parallel_loop(0, HB // 16, unroll=8)
        def zb(i):
            hist_v[pl.ds(i * 16, 16)] = zero16i

        @plsc.parallel_loop(0, NV, unroll=8)
        def h1(i):
            v = row_v[pl.ds(i * 16, 16)]
            keys = plsc.bitcast(v, jnp.int32) & jnp.int32(0x7FFFFFFF)
            plsc.addupdate_scatter(hist_v, (keys >> SHIFT,), ones16)

        def hcnt(b):
            return _scalar(hist_v[pl.ds(b, 16)])

        def wcond(bc):
            b, acc = bc
            return acc + hcnt(b) < K

        def wbody(bc):
            b, acc = bc
            return b - 1, acc + hcnt(b)

        bstar, acc = lax.while_loop(wcond, wbody, (jnp.int32(HB - 1), jnp.int32(0)))
        k2 = K - acc

        @plsc.parallel_loop(0, NV, unroll=8, carry=jnp.int32(0))
        def cp(i, off):
            v = row_v[pl.ds(i * 16, 16)]
            keys = plsc.bitcast(v, jnp.int32) & jnp.int32(0x7FFFFFFF)
            m = (keys >> SHIFT) == bstar
            plsc.store_compressed(cand_v.at[pl.ds(off, 16)], keys, mask=m)
            return off + _scalar(plsc.all_reduce_population_count(m))
        ncand = cp
        cand_v[pl.ds(ncand, 16)] = zero16i

        nvc = (ncand + 15) >> 4

        def bit_step(j, lo):
            mid = lo | (jnp.int32(1) << (jnp.int32(SHIFT - 1) - j))

            @plsc.parallel_loop(0, nvc, unroll=4, carry=zero16i)
            def av(i, a):
                vk = cand_v[pl.ds(i * 16, 16)]
                return a + jnp.where(vk >= mid, 1, 0)
            return jnp.where(jnp.sum(av) >= k2, mid, lo)
        thr = lax.fori_loop(0, SHIFT, bit_step, bstar << SHIFT)

        @plsc.parallel_loop(0, NV, unroll=8)
        def mp(i):
            v = row_v[pl.ds(i * 16, 16)]
            keys = plsc.bitcast(v, jnp.int32) & jnp.int32(0x7FFFFFFF)
            row_v[pl.ds(i * 16, 16)] = jnp.where(keys >= thr, v, zero16f)

        pltpu.sync_copy(row_v, o_hbm.at[row])
        return carry

    lax.fori_loop(0, RPW, do_row, 0)


def kernel(input):
    x = input
    B, C, _ = x.shape
    out = _sc_topk(x.reshape(ROWS, W))
    return out.reshape(B, C, W)
